# Initial kernel scaffold; baseline (speedup 1.0000x reference)
import jax
import jax.numpy as jnp
from jax.experimental import pallas as pl


def kernel(features, conv_w, conv_b, logit_w, logit_b, delta_w, delta_b, reference_frame_idx):
    raise NotImplementedError("write your pallas kernel here")


# speedup vs baseline: 6.7866x; 6.7866x over previous
import math

import jax
import jax.numpy as jnp
from jax import lax
from jax.experimental import pallas as pl
from jax.experimental.pallas import tpu as pltpu

NFR = 5
CHN = 256
HSP = 50
NANC = 3
PRE = 1000
POST = 300
THR = 0.7
IMGSZ = 800.0
SCLAMP = math.log(1000.0 / 16.0)
SIZES = (32.0, 64.0, 128.0)

PW = HSP + 2
PF = PW * PW
PFP = 2816
XW = 2944
KP = 1024
TP = 384
NEG = -3.0e38
HI = jax.lax.Precision.HIGHEST
DEF = jax.lax.Precision.DEFAULT


def _cumsum_lanes(x):
    n = x.shape[-1]
    sh = 1
    while sh < n:
        shifted = jnp.concatenate(
            [jnp.zeros((x.shape[0], sh), x.dtype), x[:, :-sh]], axis=1)
        x = x + shifted
        sh *= 2
    return x


def _head_body(xp_ref, w9_ref, wh_ref, cb_ref, bh_ref, lg_ref, bx_ref):
    x = xp_ref[0]
    acc = jnp.zeros((CHN, PFP), jnp.float32)
    for t in range(9):
        dy, dx = divmod(t, 3)
        off = dy * PW + dx
        acc = acc + lax.dot(w9_ref[t], x[:, off:off + PFP],
                            precision=DEF, preferred_element_type=jnp.float32)
    tact = jnp.maximum(acc + cb_ref[:, :1], 0.0)
    heads = lax.dot(wh_ref[...], tact, precision=DEF,
                    preferred_element_type=jnp.float32) + bh_ref[:, :1]
    lg_ref[0] = heads[:8]
    iv = lax.broadcasted_iota(jnp.int32, (1, PFP), 1)
    xb = iv % PW
    yb = iv // PW
    cx = (xb.astype(jnp.float32) + 0.5) * 16.0
    cy = (yb.astype(jnp.float32) + 0.5) * 16.0
    rows = []
    for a, sz in enumerate(SIZES):
        dxv = heads[3 + 4 * a:4 + 4 * a]
        dyv = heads[4 + 4 * a:5 + 4 * a]
        dwv = jnp.minimum(heads[5 + 4 * a:6 + 4 * a], SCLAMP)
        dhv = jnp.minimum(heads[6 + 4 * a:7 + 4 * a], SCLAMP)
        px = dxv * sz + cx
        py = dyv * sz + cy
        hw = jnp.exp(dwv) * (sz * 0.5)
        hh = jnp.exp(dhv) * (sz * 0.5)
        rows += [jnp.clip(px - hw, 0.0, IMGSZ), jnp.clip(py - hh, 0.0, IMGSZ),
                 jnp.clip(px + hw, 0.0, IMGSZ), jnp.clip(py + hh, 0.0, IMGSZ)]
    rows.append(jnp.zeros((4, PFP), jnp.float32))
    bx_ref[0] = jnp.concatenate(rows, axis=0)


def _select_body(rf_ref, lg_ref, bx_ref, tb_ref, ts_ref, m_ref):
    ridx = rf_ref[0]
    sc = jnp.zeros((NANC, PFP), jnp.float32)
    bref = jnp.zeros((16, PFP), jnp.float32)
    for f in range(NFR):
        w = jnp.where(ridx == f, 1.0, 0.0)
        sc = sc + w * lg_ref[f, :NANC]
        bref = bref + w * bx_ref[f]
    iv = lax.broadcasted_iota(jnp.int32, (1, PFP), 1)
    xb = iv % PW
    yb = iv // PW
    valid = (xb < HSP) & (yb < HSP)
    pflat = yb * HSP + xb
    s3 = jnp.where(valid, sc, NEG)
    ai = lax.broadcasted_iota(jnp.int32, (NANC, PFP), 0)
    jidx3 = pflat * NANC + ai
    bits = lax.bitcast_convert_type(s3, jnp.uint32)
    key = bits ^ jnp.where(bits >> 31 > 0,
                           jnp.uint32(0xFFFFFFFF), jnp.uint32(0x80000000))

    def bitstep(b, tcur):
        cand = tcur | (jnp.uint32(1) << (jnp.uint32(31) - b.astype(jnp.uint32)))
        cnt = jnp.sum(jnp.where(key >= cand, 1.0, 0.0))
        return jnp.where(cnt >= PRE, cand, tcur)

    tstar = lax.fori_loop(0, 32, bitstep, jnp.uint32(0))
    gt = key > tstar
    eq = key == tstar
    n_gt = jnp.sum(jnp.where(gt, 1.0, 0.0))
    quota = PRE - n_gt
    eqf = jnp.where(eq, 1.0, 0.0)
    colcnt = eqf[0:1] + eqf[1:2] + eqf[2:3]
    ex = _cumsum_lanes(colcnt) - colcnt
    tie0 = ex
    tie1 = ex + eqf[0:1]
    tie2 = tie1 + eqf[1:2]
    tiepos = jnp.concatenate([tie0, tie1, tie2], axis=0)
    sel = gt | (eq & (tiepos < quota))
    self_ = jnp.where(sel, 1.0, 0.0)
    colsel = self_[0:1] + self_[1:2] + self_[2:3]
    exs = _cumsum_lanes(colsel) - colsel
    pos0 = exs
    pos1 = exs + self_[0:1]
    pos2 = pos1 + self_[1:2]
    pos = jnp.concatenate([pos0, pos1, pos2], axis=0)
    pos = jnp.where(sel, pos, -1.0)
    jidx3f = jidx3.astype(jnp.float32)
    cmat = jnp.zeros((6, KP), jnp.float32)
    rio = lax.broadcasted_iota(jnp.int32, (704, KP), 1).astype(jnp.float32)
    for a in range(NANC):
        for c0 in range(0, PFP, 704):
            posc = jnp.transpose(pos[a:a + 1, c0:c0 + 704])
            ot = jnp.where(posc == rio, 1.0, 0.0)
            vc = jnp.concatenate(
                [s3[a:a + 1, c0:c0 + 704], jidx3f[a:a + 1, c0:c0 + 704],
                 bref[4 * a:4 * a + 4, c0:c0 + 704]], axis=0)
            cmat = cmat + lax.dot(vc, ot, precision=HI,
                                  preferred_element_type=jnp.float32)
    rl = lax.broadcasted_iota(jnp.int32, (1, KP), 1)
    isr = rl < PRE
    svec = jnp.where(isr, cmat[0:1], NEG)
    jvec = jnp.where(isr, cmat[1:2], 30000.0 + rl.astype(jnp.float32))
    scol = jnp.transpose(svec)
    jcol = jnp.transpose(jvec)
    srow = svec
    cmp = (jnp.where(srow > scol, 1.0, 0.0)
           + jnp.where((srow == scol) & (jvec < jcol), 1.0, 0.0))
    rank = jnp.sum(cmp, axis=1, keepdims=True)
    rio2 = lax.broadcasted_iota(jnp.int32, (KP, KP), 1).astype(jnp.float32)
    ot2 = jnp.where(rank == rio2, 1.0, 0.0)
    cfix = jnp.concatenate([svec, jvec, cmat[2:6]], axis=0)
    smat = lax.dot(cfix, ot2, precision=HI,
                   preferred_element_type=jnp.float32)
    x0r, y0r, x1r, y1r = (smat[2:3], smat[3:4], smat[4:5], smat[5:6])
    area_r = (x1r - x0r) * (y1r - y0r)
    x0c = jnp.transpose(x0r)
    y0c = jnp.transpose(y0r)
    x1c = jnp.transpose(x1r)
    y1c = jnp.transpose(y1r)
    area_c = jnp.transpose(area_r)
    for cb in range(4):
        sl = slice(cb * 256, cb * 256 + 256)
        ltx = jnp.maximum(x0c[sl], x0r)
        lty = jnp.maximum(y0c[sl], y0r)
        rbx = jnp.minimum(x1c[sl], x1r)
        rby = jnp.minimum(y1c[sl], y1r)
        ww = jnp.clip(rbx - ltx, 0.0, None)
        hh = jnp.clip(rby - lty, 0.0, None)
        inter = ww * hh
        iou = inter / (area_c[sl] + area_r - inter + 1e-9)
        m_ref[cb * 256:cb * 256 + 256, :] = jnp.where(iou > THR, 1.0, 0.0)
    lanes = lax.broadcasted_iota(jnp.int32, (1, KP), 1)

    def nms_step(i, supp):
        row = m_ref[pl.ds(i, 1), :]
        onehot = jnp.where(lanes == i, 1.0, 0.0)
        alive = 1.0 - jnp.sum(supp * onehot)
        upd = row * jnp.where(lanes > i, 1.0, 0.0) * alive
        return jnp.maximum(supp, upd)

    supp = lax.fori_loop(0, PRE, nms_step, jnp.zeros((1, KP), jnp.float32))
    alivev = (1.0 - supp) * jnp.where(lanes < PRE, 1.0, 0.0)
    posk = _cumsum_lanes(alivev) - 1.0
    poskm = jnp.where(alivev > 0, posk, -1.0)
    n_alive = jnp.sum(alivev)
    poskc = jnp.transpose(poskm)
    tio = lax.broadcasted_iota(jnp.int32, (KP, TP), 1).astype(jnp.float32)
    rc = lax.broadcasted_iota(jnp.int32, (KP, TP), 0)
    kt = jnp.where(poskc == tio, 1.0, 0.0)
    kt = kt + jnp.where((tio >= n_alive) & (rc == 0), 1.0, 0.0)
    kept = lax.dot(smat[0:2], kt, precision=HI,
                   preferred_element_type=jnp.float32)
    ts_ref[...] = jnp.concatenate(
        [kept, jnp.zeros((6, TP), jnp.float32)], axis=0)
    jk = kept[1:2].astype(jnp.int32)
    pk = jk // NANC
    ak = jk % NANC
    lane_t = (pk // HSP) * PW + (pk % HSP)
    gsub = lax.broadcasted_iota(jnp.int32, (PFP, TP), 0)
    frames = [jnp.zeros((4, TP), jnp.float32) for _ in range(NFR)]
    for a in range(NANC):
        ga = jnp.where((gsub == lane_t) & (ak == a), 1.0, 0.0)
        for f in range(NFR):
            frames[f] = frames[f] + lax.dot(
                bx_ref[f, 4 * a:4 * a + 4], ga, precision=HI,
                preferred_element_type=jnp.float32)
    out = jnp.concatenate(
        [jnp.concatenate([fr, jnp.zeros((4, TP), jnp.float32)], axis=0)[None]
         for fr in frames], axis=0)
    tb_ref[...] = out


def kernel(features, conv_w, conv_b, logit_w, logit_b, delta_w, delta_b,
           reference_frame_idx):
    f32 = jnp.float32
    xp = jnp.pad(features, ((0, 0), (0, 0), (1, 1), (1, 1)))
    xp = xp.reshape(NFR, CHN, PF)
    xp = jnp.pad(xp, ((0, 0), (0, 0), (0, XW - PF)))
    w9 = jnp.transpose(conv_w, (2, 3, 0, 1)).reshape(9, CHN, CHN)
    wh = jnp.concatenate([logit_w[:, :, 0, 0], delta_w[:, :, 0, 0],
                          jnp.zeros((1, CHN), f32)], axis=0)
    bh = jnp.concatenate([logit_b, delta_b, jnp.zeros((1,), f32)])
    cb2 = conv_b.reshape(CHN, 1)
    bh2 = bh.reshape(16, 1)

    lg, bx = pl.pallas_call(
        _head_body,
        grid=(NFR,),
        in_specs=[
            pl.BlockSpec((1, CHN, XW), lambda f: (f, 0, 0)),
            pl.BlockSpec((9, CHN, CHN), lambda f: (0, 0, 0)),
            pl.BlockSpec((16, CHN), lambda f: (0, 0)),
            pl.BlockSpec((CHN, 1), lambda f: (0, 0)),
            pl.BlockSpec((16, 1), lambda f: (0, 0)),
        ],
        out_specs=[
            pl.BlockSpec((1, 8, PFP), lambda f: (f, 0, 0)),
            pl.BlockSpec((1, 16, PFP), lambda f: (f, 0, 0)),
        ],
        out_shape=[
            jax.ShapeDtypeStruct((NFR, 8, PFP), f32),
            jax.ShapeDtypeStruct((NFR, 16, PFP), f32),
        ],
    )(xp, w9, wh, cb2, bh2)

    rf = jnp.asarray(reference_frame_idx, jnp.int32).reshape(1)
    tb, ts = pl.pallas_call(
        _select_body,
        grid_spec=pltpu.PrefetchScalarGridSpec(
            num_scalar_prefetch=1,
            grid=(1,),
            in_specs=[
                pl.BlockSpec((NFR, 8, PFP), lambda i, s: (0, 0, 0)),
                pl.BlockSpec((NFR, 16, PFP), lambda i, s: (0, 0, 0)),
            ],
            out_specs=[
                pl.BlockSpec((NFR, 8, TP), lambda i, s: (0, 0, 0)),
                pl.BlockSpec((8, TP), lambda i, s: (0, 0)),
            ],
            scratch_shapes=[pltpu.VMEM((KP, KP), f32)],
        ),
        out_shape=[
            jax.ShapeDtypeStruct((NFR, 8, TP), f32),
            jax.ShapeDtypeStruct((8, TP), f32),
        ],
    )(rf, lg, bx)

    def interior(x):
        lead = x.shape[:-1]
        y = x[..., :PF].reshape(*lead, PW, PW)[..., :HSP, :HSP]
        return y.reshape(*lead, HSP * HSP)

    logits_flat = jnp.transpose(interior(lg[:, :NANC]), (0, 2, 1)
                                ).reshape(NFR, -1)
    props = jnp.transpose(
        interior(bx[:, :12]).reshape(NFR, NANC, 4, HSP * HSP),
        (0, 3, 1, 2)).reshape(NFR, -1, 4)
    tubelet_boxes = jnp.transpose(tb[:, :4, :POST], (0, 2, 1))
    tubelet_scores = ts[0, :POST]
    return (tubelet_boxes, tubelet_scores, props, logits_flat)


# speedup vs baseline: 7.1105x; 1.0477x over previous
import math

import jax
import jax.numpy as jnp
from jax import lax
from jax.experimental import pallas as pl
from jax.experimental.pallas import tpu as pltpu

NFR = 5
CHN = 256
HSP = 50
NANC = 3
PRE = 1000
POST = 300
THR = 0.7
IMGSZ = 800.0
SCLAMP = math.log(1000.0 / 16.0)
SIZES = (32.0, 64.0, 128.0)

PW = HSP + 2
PF = PW * PW
PFP = 2816
XW = 2944
KP = 1024
TP = 384
NEG = -3.0e38
HI = jax.lax.Precision.HIGHEST
DEF = jax.lax.Precision.DEFAULT


def _cumsum_lanes(x):
    n = x.shape[-1]
    sh = 1
    while sh < n:
        shifted = jnp.concatenate(
            [jnp.zeros((x.shape[0], sh), x.dtype), x[:, :-sh]], axis=1)
        x = x + shifted
        sh *= 2
    return x


def _head_body(xp_ref, w9_ref, wh_ref, cb_ref, bh_ref, lg_ref, bx_ref):
    x = xp_ref[0]
    acc = jnp.zeros((CHN, PFP), jnp.float32)
    for t in range(9):
        dy, dx = divmod(t, 3)
        off = dy * PW + dx
        acc = acc + lax.dot(w9_ref[t], x[:, off:off + PFP],
                            precision=DEF, preferred_element_type=jnp.float32)
    tact = jnp.maximum(acc + cb_ref[:, :1], 0.0)
    heads = lax.dot(wh_ref[...], tact, precision=DEF,
                    preferred_element_type=jnp.float32) + bh_ref[:, :1]
    lg_ref[0] = heads[:8]
    iv = lax.broadcasted_iota(jnp.int32, (1, PFP), 1)
    xb = iv % PW
    yb = iv // PW
    cx = (xb.astype(jnp.float32) + 0.5) * 16.0
    cy = (yb.astype(jnp.float32) + 0.5) * 16.0
    rows = []
    for a, sz in enumerate(SIZES):
        dxv = heads[3 + 4 * a:4 + 4 * a]
        dyv = heads[4 + 4 * a:5 + 4 * a]
        dwv = jnp.minimum(heads[5 + 4 * a:6 + 4 * a], SCLAMP)
        dhv = jnp.minimum(heads[6 + 4 * a:7 + 4 * a], SCLAMP)
        px = dxv * sz + cx
        py = dyv * sz + cy
        hw = jnp.exp(dwv) * (sz * 0.5)
        hh = jnp.exp(dhv) * (sz * 0.5)
        rows += [jnp.clip(px - hw, 0.0, IMGSZ), jnp.clip(py - hh, 0.0, IMGSZ),
                 jnp.clip(px + hw, 0.0, IMGSZ), jnp.clip(py + hh, 0.0, IMGSZ)]
    rows.append(jnp.zeros((4, PFP), jnp.float32))
    bx_ref[0] = jnp.concatenate(rows, axis=0)


def _select_body(rf_ref, lg_ref, bx_ref, tb_ref, ts_ref, m_ref):
    ridx = rf_ref[0]
    sc = jnp.zeros((NANC, PFP), jnp.float32)
    bref = jnp.zeros((16, PFP), jnp.float32)
    for f in range(NFR):
        w = jnp.where(ridx == f, 1.0, 0.0)
        sc = sc + w * lg_ref[f, :NANC]
        bref = bref + w * bx_ref[f]
    iv = lax.broadcasted_iota(jnp.int32, (1, PFP), 1)
    xb = iv % PW
    yb = iv // PW
    valid = (xb < HSP) & (yb < HSP)
    pflat = yb * HSP + xb
    s3 = jnp.where(valid, sc, NEG)
    ai = lax.broadcasted_iota(jnp.int32, (NANC, PFP), 0)
    jidx3 = pflat * NANC + ai
    bits = lax.bitcast_convert_type(s3, jnp.uint32)
    key = bits ^ jnp.where(bits >> 31 > 0,
                           jnp.uint32(0xFFFFFFFF), jnp.uint32(0x80000000))

    def bitstep(b, tcur):
        cand = tcur | (jnp.uint32(1) << (jnp.uint32(31) - b.astype(jnp.uint32)))
        cnt = jnp.sum(jnp.where(key >= cand, 1.0, 0.0))
        return jnp.where(cnt >= PRE, cand, tcur)

    tstar = lax.fori_loop(0, 32, bitstep, jnp.uint32(0))
    gt = key > tstar
    eq = key == tstar
    n_gt = jnp.sum(jnp.where(gt, 1.0, 0.0))
    quota = PRE - n_gt
    eqf = jnp.where(eq, 1.0, 0.0)
    colcnt = eqf[0:1] + eqf[1:2] + eqf[2:3]
    ex = _cumsum_lanes(colcnt) - colcnt
    tie0 = ex
    tie1 = ex + eqf[0:1]
    tie2 = tie1 + eqf[1:2]
    tiepos = jnp.concatenate([tie0, tie1, tie2], axis=0)
    sel = gt | (eq & (tiepos < quota))
    self_ = jnp.where(sel, 1.0, 0.0)
    colsel = self_[0:1] + self_[1:2] + self_[2:3]
    exs = _cumsum_lanes(colsel) - colsel
    pos0 = exs
    pos1 = exs + self_[0:1]
    pos2 = pos1 + self_[1:2]
    pos = jnp.concatenate([pos0, pos1, pos2], axis=0)
    pos = jnp.where(sel, pos, -1.0)
    jidx3f = jidx3.astype(jnp.float32)
    cmat = jnp.zeros((6, KP), jnp.float32)
    rio = lax.broadcasted_iota(jnp.int32, (704, KP), 1).astype(jnp.float32)
    for a in range(NANC):
        for c0 in range(0, PFP, 704):
            posc = jnp.transpose(pos[a:a + 1, c0:c0 + 704])
            ot = jnp.where(posc == rio, 1.0, 0.0)
            vc = jnp.concatenate(
                [s3[a:a + 1, c0:c0 + 704], jidx3f[a:a + 1, c0:c0 + 704],
                 bref[4 * a:4 * a + 4, c0:c0 + 704]], axis=0)
            cmat = cmat + lax.dot(vc, ot, precision=HI,
                                  preferred_element_type=jnp.float32)
    rl = lax.broadcasted_iota(jnp.int32, (1, KP), 1)
    isr = rl < PRE
    svec = jnp.where(isr, cmat[0:1], NEG)
    jvec = jnp.where(isr, cmat[1:2], 30000.0 + rl.astype(jnp.float32))
    scol = jnp.transpose(svec)
    jcol = jnp.transpose(jvec)
    srow = svec
    cmp = (jnp.where(srow > scol, 1.0, 0.0)
           + jnp.where((srow == scol) & (jvec < jcol), 1.0, 0.0))
    rank = jnp.sum(cmp, axis=1, keepdims=True)
    rio2 = lax.broadcasted_iota(jnp.int32, (KP, KP), 1).astype(jnp.float32)
    ot2 = jnp.where(rank == rio2, 1.0, 0.0)
    cfix = jnp.concatenate([svec, jvec, cmat[2:6]], axis=0)
    smat = lax.dot(cfix, ot2, precision=HI,
                   preferred_element_type=jnp.float32)
    x0r, y0r, x1r, y1r = (smat[2:3], smat[3:4], smat[4:5], smat[5:6])
    area_r = (x1r - x0r) * (y1r - y0r)
    x0c = jnp.transpose(x0r)
    y0c = jnp.transpose(y0r)
    x1c = jnp.transpose(x1r)
    y1c = jnp.transpose(y1r)
    area_c = jnp.transpose(area_r)
    for cb in range(4):
        sl = slice(cb * 256, cb * 256 + 256)
        ltx = jnp.maximum(x0c[sl], x0r)
        lty = jnp.maximum(y0c[sl], y0r)
        rbx = jnp.minimum(x1c[sl], x1r)
        rby = jnp.minimum(y1c[sl], y1r)
        ww = jnp.clip(rbx - ltx, 0.0, None)
        hh = jnp.clip(rby - lty, 0.0, None)
        inter = ww * hh
        iou = inter / (area_c[sl] + area_r - inter + 1e-9)
        m_ref[cb * 256:cb * 256 + 256, :] = jnp.where(iou > THR, 1.0, 0.0)
    lanes = lax.broadcasted_iota(jnp.int32, (1, KP), 1)
    lane128 = lax.broadcasted_iota(jnp.int32, (1, 128), 1)
    supp = jnp.zeros((1, KP), jnp.float32)
    for b in range(KP // 128):
        base = b * 128

        def group(g, sb, base=base):
            off = pl.multiple_of(base + g * 8, 8)
            blk = m_ref[pl.ds(off, 8), base:base + 128]
            for r in range(8):
                i = g * 8 + r
                row = blk[r:r + 1]
                onehot = jnp.where(lane128 == i, 1.0, 0.0)
                alive = 1.0 - jnp.sum(sb * onehot)
                sb = jnp.maximum(
                    sb, row * jnp.where(lane128 > i, 1.0, 0.0) * alive)
            return sb

        nblk = min(128, PRE - base)
        if nblk <= 0:
            break
        sblk = lax.fori_loop(0, nblk // 8, group, supp[:, base:base + 128])
        padded = jnp.pad(sblk, ((0, 0), (base, KP - base - 128)))
        inblk = (lanes >= base) & (lanes < base + 128)
        supp = jnp.where(inblk, padded, supp)
        if base + 128 < PRE:
            rows_m = m_ref[base:base + 128, :]
            alivec = jnp.transpose(1.0 - sblk)
            contrib = jnp.max(rows_m * alivec, axis=0, keepdims=True)
            later = jnp.where(lanes >= base + 128, 1.0, 0.0)
            supp = jnp.maximum(supp, contrib * later)
    alivev = (1.0 - supp) * jnp.where(lanes < PRE, 1.0, 0.0)
    posk = _cumsum_lanes(alivev) - 1.0
    poskm = jnp.where(alivev > 0, posk, -1.0)
    n_alive = jnp.sum(alivev)
    poskc = jnp.transpose(poskm)
    tio = lax.broadcasted_iota(jnp.int32, (KP, TP), 1).astype(jnp.float32)
    rc = lax.broadcasted_iota(jnp.int32, (KP, TP), 0)
    kt = jnp.where(poskc == tio, 1.0, 0.0)
    kt = kt + jnp.where((tio >= n_alive) & (rc == 0), 1.0, 0.0)
    kept = lax.dot(smat[0:2], kt, precision=HI,
                   preferred_element_type=jnp.float32)
    ts_ref[...] = jnp.concatenate(
        [kept, jnp.zeros((6, TP), jnp.float32)], axis=0)
    jk = kept[1:2].astype(jnp.int32)
    pk = jk // NANC
    ak = jk % NANC
    lane_t = (pk // HSP) * PW + (pk % HSP)
    gsub = lax.broadcasted_iota(jnp.int32, (PFP, TP), 0)
    frames = [jnp.zeros((4, TP), jnp.float32) for _ in range(NFR)]
    for a in range(NANC):
        ga = jnp.where((gsub == lane_t) & (ak == a), 1.0, 0.0)
        for f in range(NFR):
            frames[f] = frames[f] + lax.dot(
                bx_ref[f, 4 * a:4 * a + 4], ga, precision=HI,
                preferred_element_type=jnp.float32)
    out = jnp.concatenate(
        [jnp.concatenate([fr, jnp.zeros((4, TP), jnp.float32)], axis=0)[None]
         for fr in frames], axis=0)
    tb_ref[...] = out


def kernel(features, conv_w, conv_b, logit_w, logit_b, delta_w, delta_b,
           reference_frame_idx):
    f32 = jnp.float32
    xp = jnp.pad(features, ((0, 0), (0, 0), (1, 1), (1, 1)))
    xp = xp.reshape(NFR, CHN, PF)
    xp = jnp.pad(xp, ((0, 0), (0, 0), (0, XW - PF)))
    w9 = jnp.transpose(conv_w, (2, 3, 0, 1)).reshape(9, CHN, CHN)
    wh = jnp.concatenate([logit_w[:, :, 0, 0], delta_w[:, :, 0, 0],
                          jnp.zeros((1, CHN), f32)], axis=0)
    bh = jnp.concatenate([logit_b, delta_b, jnp.zeros((1,), f32)])
    cb2 = conv_b.reshape(CHN, 1)
    bh2 = bh.reshape(16, 1)

    lg, bx = pl.pallas_call(
        _head_body,
        grid=(NFR,),
        in_specs=[
            pl.BlockSpec((1, CHN, XW), lambda f: (f, 0, 0)),
            pl.BlockSpec((9, CHN, CHN), lambda f: (0, 0, 0)),
            pl.BlockSpec((16, CHN), lambda f: (0, 0)),
            pl.BlockSpec((CHN, 1), lambda f: (0, 0)),
            pl.BlockSpec((16, 1), lambda f: (0, 0)),
        ],
        out_specs=[
            pl.BlockSpec((1, 8, PFP), lambda f: (f, 0, 0)),
            pl.BlockSpec((1, 16, PFP), lambda f: (f, 0, 0)),
        ],
        out_shape=[
            jax.ShapeDtypeStruct((NFR, 8, PFP), f32),
            jax.ShapeDtypeStruct((NFR, 16, PFP), f32),
        ],
    )(xp, w9, wh, cb2, bh2)

    rf = jnp.asarray(reference_frame_idx, jnp.int32).reshape(1)
    tb, ts = pl.pallas_call(
        _select_body,
        grid_spec=pltpu.PrefetchScalarGridSpec(
            num_scalar_prefetch=1,
            grid=(1,),
            in_specs=[
                pl.BlockSpec((NFR, 8, PFP), lambda i, s: (0, 0, 0)),
                pl.BlockSpec((NFR, 16, PFP), lambda i, s: (0, 0, 0)),
            ],
            out_specs=[
                pl.BlockSpec((NFR, 8, TP), lambda i, s: (0, 0, 0)),
                pl.BlockSpec((8, TP), lambda i, s: (0, 0)),
            ],
            scratch_shapes=[pltpu.VMEM((KP, KP), f32)],
        ),
        out_shape=[
            jax.ShapeDtypeStruct((NFR, 8, TP), f32),
            jax.ShapeDtypeStruct((8, TP), f32),
        ],
    )(rf, lg, bx)

    def interior(x):
        lead = x.shape[:-1]
        y = x[..., :PF].reshape(*lead, PW, PW)[..., :HSP, :HSP]
        return y.reshape(*lead, HSP * HSP)

    logits_flat = jnp.transpose(interior(lg[:, :NANC]), (0, 2, 1)
                                ).reshape(NFR, -1)
    props = jnp.transpose(
        interior(bx[:, :12]).reshape(NFR, NANC, 4, HSP * HSP),
        (0, 3, 1, 2)).reshape(NFR, -1, 4)
    tubelet_boxes = jnp.transpose(tb[:, :4, :POST], (0, 2, 1))
    tubelet_scores = ts[0, :POST]
    return (tubelet_boxes, tubelet_scores, props, logits_flat)


# speedup vs baseline: 7.1405x; 1.0042x over previous
import math

import jax
import jax.numpy as jnp
from jax import lax
from jax.experimental import pallas as pl
from jax.experimental.pallas import tpu as pltpu

NFR = 5
CHN = 256
HSP = 50
NANC = 3
PRE = 1000
POST = 300
THR = 0.7
IMGSZ = 800.0
SCLAMP = math.log(1000.0 / 16.0)
SIZES = (32.0, 64.0, 128.0)

PW = HSP + 2
PF = PW * PW
PFP = 2816
XW = 2944
KP = 1024
TP = 384
NEG = -3.0e38
HI = jax.lax.Precision.HIGHEST
DEF = jax.lax.Precision.DEFAULT


def _cumsum_lanes(x):
    n = x.shape[-1]
    sh = 1
    while sh < n:
        shifted = jnp.concatenate(
            [jnp.zeros((x.shape[0], sh), x.dtype), x[:, :-sh]], axis=1)
        x = x + shifted
        sh *= 2
    return x


def _head_body(xp_ref, w9_ref, wh_ref, cb_ref, bh_ref, lg_ref, bx_ref):
    acc = jnp.zeros((CHN, PFP), jnp.float32)
    for t in range(9):
        dy, dx = divmod(t, 3)
        off = dy * PW + dx
        acc = acc + lax.dot(w9_ref[t], xp_ref[0, :, off:off + PFP],
                            precision=DEF, preferred_element_type=jnp.float32)
    tact = jnp.maximum(acc + cb_ref[:, :1], 0.0)
    heads = lax.dot(wh_ref[...], tact, precision=DEF,
                    preferred_element_type=jnp.float32) + bh_ref[:, :1]
    lg_ref[0] = heads[:8]
    iv = lax.broadcasted_iota(jnp.int32, (1, PFP), 1)
    xb = iv % PW
    yb = iv // PW
    cx = (xb.astype(jnp.float32) + 0.5) * 16.0
    cy = (yb.astype(jnp.float32) + 0.5) * 16.0
    rows = []
    for a, sz in enumerate(SIZES):
        dxv = heads[3 + 4 * a:4 + 4 * a]
        dyv = heads[4 + 4 * a:5 + 4 * a]
        dwv = jnp.minimum(heads[5 + 4 * a:6 + 4 * a], SCLAMP)
        dhv = jnp.minimum(heads[6 + 4 * a:7 + 4 * a], SCLAMP)
        px = dxv * sz + cx
        py = dyv * sz + cy
        hw = jnp.exp(dwv) * (sz * 0.5)
        hh = jnp.exp(dhv) * (sz * 0.5)
        rows += [jnp.clip(px - hw, 0.0, IMGSZ), jnp.clip(py - hh, 0.0, IMGSZ),
                 jnp.clip(px + hw, 0.0, IMGSZ), jnp.clip(py + hh, 0.0, IMGSZ)]
    rows.append(jnp.zeros((4, PFP), jnp.float32))
    bx_ref[0] = jnp.concatenate(rows, axis=0)


def _select_body(rf_ref, lg_ref, bx_ref, tb_ref, ts_ref, m_ref):
    ridx = rf_ref[0]
    sc = jnp.zeros((NANC, PFP), jnp.float32)
    bref = jnp.zeros((16, PFP), jnp.float32)
    for f in range(NFR):
        w = jnp.where(ridx == f, 1.0, 0.0)
        sc = sc + w * lg_ref[f, :NANC]
        bref = bref + w * bx_ref[f]
    iv = lax.broadcasted_iota(jnp.int32, (1, PFP), 1)
    xb = iv % PW
    yb = iv // PW
    valid = (xb < HSP) & (yb < HSP)
    pflat = yb * HSP + xb
    s3 = jnp.where(valid, sc, NEG)
    ai = lax.broadcasted_iota(jnp.int32, (NANC, PFP), 0)
    jidx3 = pflat * NANC + ai
    bits = lax.bitcast_convert_type(s3, jnp.uint32)
    key = bits ^ jnp.where(bits >> 31 > 0,
                           jnp.uint32(0xFFFFFFFF), jnp.uint32(0x80000000))

    def bitstep(b, tcur):
        cand = tcur | (jnp.uint32(1) << (jnp.uint32(31) - b.astype(jnp.uint32)))
        cnt = jnp.sum(jnp.where(key >= cand, 1.0, 0.0))
        return jnp.where(cnt >= PRE, cand, tcur)

    tstar = lax.fori_loop(0, 32, bitstep, jnp.uint32(0))
    gt = key > tstar
    eq = key == tstar
    n_gt = jnp.sum(jnp.where(gt, 1.0, 0.0))
    quota = PRE - n_gt
    eqf = jnp.where(eq, 1.0, 0.0)
    colcnt = eqf[0:1] + eqf[1:2] + eqf[2:3]
    ex = _cumsum_lanes(colcnt) - colcnt
    tie0 = ex
    tie1 = ex + eqf[0:1]
    tie2 = tie1 + eqf[1:2]
    tiepos = jnp.concatenate([tie0, tie1, tie2], axis=0)
    sel = gt | (eq & (tiepos < quota))
    self_ = jnp.where(sel, 1.0, 0.0)
    colsel = self_[0:1] + self_[1:2] + self_[2:3]
    exs = _cumsum_lanes(colsel) - colsel
    pos0 = exs
    pos1 = exs + self_[0:1]
    pos2 = pos1 + self_[1:2]
    pos = jnp.concatenate([pos0, pos1, pos2], axis=0)
    pos = jnp.where(sel, pos, -1.0)
    jidx3f = jidx3.astype(jnp.float32)
    cmat = jnp.zeros((6, KP), jnp.float32)
    rio = lax.broadcasted_iota(jnp.int32, (704, KP), 1).astype(jnp.float32)
    for a in range(NANC):
        for c0 in range(0, PFP, 704):
            posc = jnp.transpose(pos[a:a + 1, c0:c0 + 704])
            ot = jnp.where(posc == rio, 1.0, 0.0)
            vc = jnp.concatenate(
                [s3[a:a + 1, c0:c0 + 704], jidx3f[a:a + 1, c0:c0 + 704],
                 bref[4 * a:4 * a + 4, c0:c0 + 704]], axis=0)
            cmat = cmat + lax.dot(vc, ot, precision=HI,
                                  preferred_element_type=jnp.float32)
    rl = lax.broadcasted_iota(jnp.int32, (1, KP), 1)
    isr = rl < PRE
    svec = jnp.where(isr, cmat[0:1], NEG)
    jvec = jnp.where(isr, cmat[1:2], 30000.0 + rl.astype(jnp.float32))
    scol = jnp.transpose(svec)
    jcol = jnp.transpose(jvec)
    srow = svec
    cmp = (jnp.where(srow > scol, 1.0, 0.0)
           + jnp.where((srow == scol) & (jvec < jcol), 1.0, 0.0))
    rank = jnp.sum(cmp, axis=1, keepdims=True)
    rio2 = lax.broadcasted_iota(jnp.int32, (KP, KP), 1).astype(jnp.float32)
    ot2 = jnp.where(rank == rio2, 1.0, 0.0)
    cfix = jnp.concatenate([svec, jvec, cmat[2:6]], axis=0)
    smat = lax.dot(cfix, ot2, precision=HI,
                   preferred_element_type=jnp.float32)
    x0r, y0r, x1r, y1r = (smat[2:3], smat[3:4], smat[4:5], smat[5:6])
    area_r = (x1r - x0r) * (y1r - y0r)
    x0c = jnp.transpose(x0r)
    y0c = jnp.transpose(y0r)
    x1c = jnp.transpose(x1r)
    y1c = jnp.transpose(y1r)
    area_c = jnp.transpose(area_r)
    for cb in range(4):
        sl = slice(cb * 256, cb * 256 + 256)
        ltx = jnp.maximum(x0c[sl], x0r)
        lty = jnp.maximum(y0c[sl], y0r)
        rbx = jnp.minimum(x1c[sl], x1r)
        rby = jnp.minimum(y1c[sl], y1r)
        ww = jnp.clip(rbx - ltx, 0.0, None)
        hh = jnp.clip(rby - lty, 0.0, None)
        inter = ww * hh
        iou = inter / (area_c[sl] + area_r - inter + 1e-9)
        m_ref[cb * 256:cb * 256 + 256, :] = jnp.where(iou > THR, 1.0, 0.0)
    lanes = lax.broadcasted_iota(jnp.int32, (1, KP), 1)
    lane128 = lax.broadcasted_iota(jnp.int32, (1, 128), 1)
    supp = jnp.zeros((1, KP), jnp.float32)
    for b in range(KP // 128):
        base = b * 128

        def group(g, sb, base=base):
            off = pl.multiple_of(base + g * 8, 8)
            blk = m_ref[pl.ds(off, 8), base:base + 128]
            for r in range(8):
                i = g * 8 + r
                row = blk[r:r + 1]
                onehot = jnp.where(lane128 == i, 1.0, 0.0)
                alive = 1.0 - jnp.sum(sb * onehot)
                sb = jnp.maximum(
                    sb, row * jnp.where(lane128 > i, 1.0, 0.0) * alive)
            return sb

        nblk = min(128, PRE - base)
        if nblk <= 0:
            break
        sblk = lax.fori_loop(0, nblk // 8, group, supp[:, base:base + 128])
        padded = jnp.pad(sblk, ((0, 0), (base, KP - base - 128)))
        inblk = (lanes >= base) & (lanes < base + 128)
        supp = jnp.where(inblk, padded, supp)
        if base + 128 < PRE:
            rows_m = m_ref[base:base + 128, :]
            alivec = jnp.transpose(1.0 - sblk)
            contrib = jnp.max(rows_m * alivec, axis=0, keepdims=True)
            later = jnp.where(lanes >= base + 128, 1.0, 0.0)
            supp = jnp.maximum(supp, contrib * later)
    alivev = (1.0 - supp) * jnp.where(lanes < PRE, 1.0, 0.0)
    posk = _cumsum_lanes(alivev) - 1.0
    poskm = jnp.where(alivev > 0, posk, -1.0)
    n_alive = jnp.sum(alivev)
    poskc = jnp.transpose(poskm)
    tio = lax.broadcasted_iota(jnp.int32, (KP, TP), 1).astype(jnp.float32)
    rc = lax.broadcasted_iota(jnp.int32, (KP, TP), 0)
    kt = jnp.where(poskc == tio, 1.0, 0.0)
    kt = kt + jnp.where((tio >= n_alive) & (rc == 0), 1.0, 0.0)
    kept = lax.dot(smat[0:2], kt, precision=HI,
                   preferred_element_type=jnp.float32)
    ts_ref[...] = jnp.concatenate(
        [kept, jnp.zeros((6, TP), jnp.float32)], axis=0)
    jk = kept[1:2].astype(jnp.int32)
    pk = jk // NANC
    ak = jk % NANC
    lane_t = (pk // HSP) * PW + (pk % HSP)
    gsub = lax.broadcasted_iota(jnp.int32, (PFP, TP), 0)
    frames = [jnp.zeros((4, TP), jnp.float32) for _ in range(NFR)]
    for a in range(NANC):
        ga = jnp.where((gsub == lane_t) & (ak == a), 1.0, 0.0)
        for f in range(NFR):
            frames[f] = frames[f] + lax.dot(
                bx_ref[f, 4 * a:4 * a + 4], ga, precision=HI,
                preferred_element_type=jnp.float32)
    out = jnp.concatenate(
        [jnp.concatenate([fr, jnp.zeros((4, TP), jnp.float32)], axis=0)[None]
         for fr in frames], axis=0)
    tb_ref[...] = out


def kernel(features, conv_w, conv_b, logit_w, logit_b, delta_w, delta_b,
           reference_frame_idx):
    f32 = jnp.float32
    xp = jnp.pad(features, ((0, 0), (0, 0), (1, 1), (1, 1)))
    xp = xp.reshape(NFR, CHN, PF)
    xp = jnp.pad(xp, ((0, 0), (0, 0), (0, XW - PF)))
    w9 = jnp.transpose(conv_w, (2, 3, 0, 1)).reshape(9, CHN, CHN)
    wh = jnp.concatenate([logit_w[:, :, 0, 0], delta_w[:, :, 0, 0],
                          jnp.zeros((1, CHN), f32)], axis=0)
    bh = jnp.concatenate([logit_b, delta_b, jnp.zeros((1,), f32)])
    cb2 = conv_b.reshape(CHN, 1)
    bh2 = bh.reshape(16, 1)

    lg, bx = pl.pallas_call(
        _head_body,
        grid=(NFR,),
        in_specs=[
            pl.BlockSpec((1, CHN, XW), lambda f: (f, 0, 0)),
            pl.BlockSpec((9, CHN, CHN), lambda f: (0, 0, 0)),
            pl.BlockSpec((16, CHN), lambda f: (0, 0)),
            pl.BlockSpec((CHN, 1), lambda f: (0, 0)),
            pl.BlockSpec((16, 1), lambda f: (0, 0)),
        ],
        out_specs=[
            pl.BlockSpec((1, 8, PFP), lambda f: (f, 0, 0)),
            pl.BlockSpec((1, 16, PFP), lambda f: (f, 0, 0)),
        ],
        out_shape=[
            jax.ShapeDtypeStruct((NFR, 8, PFP), f32),
            jax.ShapeDtypeStruct((NFR, 16, PFP), f32),
        ],
    )(xp, w9, wh, cb2, bh2)

    rf = jnp.asarray(reference_frame_idx, jnp.int32).reshape(1)
    tb, ts = pl.pallas_call(
        _select_body,
        grid_spec=pltpu.PrefetchScalarGridSpec(
            num_scalar_prefetch=1,
            grid=(1,),
            in_specs=[
                pl.BlockSpec((NFR, 8, PFP), lambda i, s: (0, 0, 0)),
                pl.BlockSpec((NFR, 16, PFP), lambda i, s: (0, 0, 0)),
            ],
            out_specs=[
                pl.BlockSpec((NFR, 8, TP), lambda i, s: (0, 0, 0)),
                pl.BlockSpec((8, TP), lambda i, s: (0, 0)),
            ],
            scratch_shapes=[pltpu.VMEM((KP, KP), f32)],
        ),
        out_shape=[
            jax.ShapeDtypeStruct((NFR, 8, TP), f32),
            jax.ShapeDtypeStruct((8, TP), f32),
        ],
    )(rf, lg, bx)

    def interior(x):
        lead = x.shape[:-1]
        y = x[..., :PF].reshape(*lead, PW, PW)[..., :HSP, :HSP]
        return y.reshape(*lead, HSP * HSP)

    logits_flat = jnp.transpose(interior(lg[:, :NANC]), (0, 2, 1)
                                ).reshape(NFR, -1)
    props = jnp.transpose(
        interior(bx[:, :12]).reshape(NFR, NANC, 4, HSP * HSP),
        (0, 3, 1, 2)).reshape(NFR, -1, 4)
    tubelet_boxes = jnp.transpose(tb[:, :4, :POST], (0, 2, 1))
    tubelet_scores = ts[0, :POST]
    return (tubelet_boxes, tubelet_scores, props, logits_flat)
